# baseline (speedup 1.0000x reference)
import functools

import jax
import jax.numpy as jnp
from jax import lax
from jax.experimental import pallas as pl
from jax.experimental.pallas import tpu as pltpu, tpu_sc as plsc

_B = 16384
_D = 32
_V = 1000000
_BLK = 131072
_NW = 32
_BPW = _B // _NW
_CHUNK = 128
_NCH = _BPW // _CHUNK
_L = 16

_sc_mesh = plsc.VectorSubcoreMesh(core_axis_name="c", subcore_axis_name="s")


def _matvec_body(u_ref, t_ref, s_ref):
    s_ref[...] = jnp.dot(
        u_ref[...], t_ref[...], preferred_element_type=jnp.float32
    ).reshape(_BLK)


def _matvec(tt, user):
    grid = (_V + _BLK - 1) // _BLK
    return pl.pallas_call(
        _matvec_body,
        grid=(grid,),
        in_specs=[
            pl.BlockSpec((1, _D), lambda i: (0, 0)),
            pl.BlockSpec((_D, _BLK), lambda i: (0, i)),
        ],
        out_specs=pl.BlockSpec((_BLK,), lambda i: (i,)),
        out_shape=jax.ShapeDtypeStruct((_V,), jnp.float32),
    )(user, tt)


@functools.partial(
    pl.kernel,
    mesh=_sc_mesh,
    out_type=jax.ShapeDtypeStruct((_B,), jnp.float32),
    compiler_params=pltpu.CompilerParams(
        needs_layout_passes=False, use_tc_tiling_on_sc=False
    ),
    scratch_types=[
        pltpu.VMEM((_NCH, _CHUNK), jnp.int32),
        pltpu.VMEM((_BPW,), jnp.float32),
        pltpu.VMEM((_BPW,), jnp.float32),
        pltpu.SemaphoreType.DMA,
    ],
)
def _gather_sigmoid(idx_hbm, s_hbm, out_hbm, idx_v, g_v, out_v, sem):
    wid = lax.axis_index("s") * 2 + lax.axis_index("c")
    base = wid * _BPW

    pltpu.sync_copy(idx_hbm.at[pl.ds(wid * _NCH, _NCH)], idx_v)

    copies = []
    for j in range(_NCH):
        copies.append(
            pltpu.async_copy(
                s_hbm.at[idx_v.at[j]], g_v.at[pl.ds(j * _CHUNK, _CHUNK)], sem
            )
        )
    for c in copies:
        c.wait()

    def body(g, carry):
        r0 = pl.multiple_of(g * _L, _L)
        x = g_v[pl.ds(r0, _L)]
        out_v[pl.ds(r0, _L)] = 1.0 / (1.0 + jnp.exp(-x))
        return carry

    lax.fori_loop(0, _BPW // _L, body, 0)

    pltpu.sync_copy(out_v, out_hbm.at[pl.ds(base, _BPW)])


def kernel(item_indices, item_table, user_table):
    tt = item_table.T
    s = _matvec(tt, user_table)
    idx = item_indices.astype(jnp.int32).reshape(_NW * _NCH, _CHUNK)
    return _gather_sigmoid(idx, s)


# speedup vs baseline: 1.0362x; 1.0362x over previous
import functools

import jax
import jax.numpy as jnp
from jax import lax
from jax.experimental import pallas as pl
from jax.experimental.pallas import tpu as pltpu, tpu_sc as plsc

_B = 16384
_D = 32
_V = 1000000
_BLK = 65536
_NW = 32
_BPW = _B // _NW
_CHUNK = 128
_NCH = _BPW // _CHUNK
_L = 16

_sc_mesh = plsc.VectorSubcoreMesh(core_axis_name="c", subcore_axis_name="s")


def _matvec_body(u_ref, t_ref, s_ref):
    s_ref[...] = jnp.dot(
        u_ref[...], t_ref[...], preferred_element_type=jnp.float32
    ).reshape(_BLK)


def _matvec(tt, user):
    grid = (_V + _BLK - 1) // _BLK
    return pl.pallas_call(
        _matvec_body,
        grid=(grid,),
        in_specs=[
            pl.BlockSpec((1, _D), lambda i: (0, 0)),
            pl.BlockSpec((_D, _BLK), lambda i: (0, i)),
        ],
        out_specs=pl.BlockSpec((_BLK,), lambda i: (i,)),
        out_shape=jax.ShapeDtypeStruct((_V,), jnp.float32),
    )(user, tt)


@functools.partial(
    pl.kernel,
    mesh=_sc_mesh,
    out_type=jax.ShapeDtypeStruct((_B,), jnp.float32),
    compiler_params=pltpu.CompilerParams(
        needs_layout_passes=False, use_tc_tiling_on_sc=False
    ),
    scratch_types=[
        pltpu.VMEM((_NCH, _CHUNK), jnp.int32),
        pltpu.VMEM((_BPW,), jnp.float32),
        pltpu.VMEM((_BPW,), jnp.float32),
        pltpu.SemaphoreType.DMA,
    ],
)
def _gather_sigmoid(idx_hbm, s_hbm, out_hbm, idx_v, g_v, out_v, sem):
    wid = lax.axis_index("s") * 2 + lax.axis_index("c")
    base = wid * _BPW

    pltpu.sync_copy(idx_hbm.at[pl.ds(wid * _NCH, _NCH)], idx_v)

    copies = []
    for j in range(_NCH):
        copies.append(
            pltpu.async_copy(
                s_hbm.at[idx_v.at[j]], g_v.at[pl.ds(j * _CHUNK, _CHUNK)], sem
            )
        )
    for c in copies:
        c.wait()

    def body(g, carry):
        r0 = pl.multiple_of(g * _L, _L)
        x = g_v[pl.ds(r0, _L)]
        out_v[pl.ds(r0, _L)] = 1.0 / (1.0 + jnp.exp(-x))
        return carry

    lax.fori_loop(0, _BPW // _L, body, 0)

    pltpu.sync_copy(out_v, out_hbm.at[pl.ds(base, _BPW)])


def kernel(item_indices, item_table, user_table):
    tt = item_table.T
    s = _matvec(tt, user_table)
    idx = item_indices.astype(jnp.int32).reshape(_NW * _NCH, _CHUNK)
    return _gather_sigmoid(idx, s)
